# baseline (speedup 1.0000x reference)
import functools

import jax
import jax.numpy as jnp
from jax import lax
from jax.experimental import pallas as pl
from jax.experimental.pallas import tpu as pltpu
from jax.experimental.pallas import tpu_sc as plsc

B = 32
CLS = 100
D = 512
CTX_LEN = 12
POOL_G = 10
POOL_A = 100
SEQ = 77
N_CTX = 36
SUF = 40
NC_SUF = 64

NCORE = 2
NSUB = 16
NW = NCORE * NSUB
G_ROWS = B * CTX_LEN
A_ROWS = 2 * B * CTX_LEN
GPT = G_ROWS // NSUB
APT = A_ROWS // NSUB
GCH = 8


def _sc_prompts(g2d_hbm, a2d_hbm, pref_hbm, suf_hbm, idxg_hbm, idxa_hbm,
                out_hbm,
                idxg_v, idxa_v, gat_v, pref_v, suf_v, ctx_sp, sem):
    cid = lax.axis_index("c")
    sid = lax.axis_index("s")
    w = sid * NCORE + cid
    s = sid

    pltpu.sync_copy(idxg_hbm.at[pl.ds(GPT * s, GPT)], idxg_v)
    pltpu.sync_copy(idxa_hbm.at[pl.ds(APT * s, APT)], idxa_v)
    for k in range(GPT // GCH):
        pltpu.async_copy(g2d_hbm.at[idxg_v.at[pl.ds(GCH * k, GCH)]],
                         gat_v, sem).wait()
        pltpu.sync_copy(gat_v, ctx_sp.at[pl.ds(GPT * s + GCH * k, GCH)])
    for k in range(APT // GCH):
        pltpu.async_copy(a2d_hbm.at[idxa_v.at[pl.ds(GCH * k, GCH)]],
                         gat_v, sem).wait()
        pltpu.sync_copy(gat_v,
                        ctx_sp.at[pl.ds(G_ROWS + APT * s + GCH * k, GCH)])

    cnt = jnp.where(w < 4, 4, 3)
    c0 = jnp.where(w < 4, 4 * w, 3 * w + 4)
    c0c = jnp.minimum(c0, CLS - 4)
    pltpu.sync_copy(pref_hbm.at[pl.ds(c0c, 4)], pref_v)
    pltpu.sync_copy(suf_hbm.at[pl.ds(c0c, 4)], suf_v)

    plsc.subcore_barrier()

    def class_body(j, carry):
        @pl.when(j < cnt)
        def _():
            c = c0 + j
            ci = c - c0c

            def batch_body(b, carry2):
                r = b * CLS + c
                pltpu.sync_copy(pref_v.at[ci], out_hbm.at[r, pl.ds(0, 1)])
                pltpu.sync_copy(ctx_sp.at[pl.ds(N_CTX * b, N_CTX)],
                                out_hbm.at[r, pl.ds(1, N_CTX)])
                pltpu.sync_copy(suf_v.at[ci], out_hbm.at[r, pl.ds(1 + N_CTX, SUF)])
                return carry2

            lax.fori_loop(0, B, batch_body, 0)
        return carry

    lax.fori_loop(0, 4, class_body, 0)


def _tok_nc_kernel(tok_ref, g_ref, ncp_ref, ncs_ref, nct_ref,
                   out_t_ref, out_ncp_ref, out_nct_ref):
    out_t_ref[...] = tok_ref[...]

    @pl.when(pl.program_id(0) == 0)
    def _write_nc():
        out_ncp_ref[...] = jnp.concatenate([
            jnp.broadcast_to(ncp_ref[...], (POOL_G, 1, D)),
            g_ref[...],
            jnp.broadcast_to(ncs_ref[...], (POOL_G, NC_SUF, D)),
        ], axis=1)
        out_nct_ref[...] = jnp.broadcast_to(nct_ref[...], (POOL_G, SEQ))


@jax.jit
def _run(idx_g, idx_a, global_prompt, attribute_prompt,
         token_prefix, token_suffix, tokenized_prompts,
         nc_token_prefix, nc_token_suffix, nc_tokenized_prompts):
    idxg_rows = (idx_g[:, None] * CTX_LEN
                 + jnp.arange(CTX_LEN, dtype=jnp.int32)[None, :]).reshape(G_ROWS)
    idxa_rows = (idx_a[:, None] * CTX_LEN
                 + jnp.arange(CTX_LEN, dtype=jnp.int32)[None, :]).reshape(A_ROWS)

    sc_fn = pl.kernel(
        _sc_prompts,
        out_type=jax.ShapeDtypeStruct((B * CLS, SEQ, D), jnp.float32),
        mesh=plsc.VectorSubcoreMesh(core_axis_name="c", subcore_axis_name="s"),
        compiler_params=pltpu.CompilerParams(use_tc_tiling_on_sc=False),
        scratch_types=[
            pltpu.VMEM((GPT,), jnp.int32),
            pltpu.VMEM((APT,), jnp.int32),
            pltpu.VMEM((GCH, D), jnp.float32),
            pltpu.VMEM((4, 1, D), jnp.float32),
            pltpu.VMEM((4, SUF, D), jnp.float32),
            pltpu.VMEM_SHARED((3 * B * CTX_LEN, D), jnp.float32),
            pltpu.SemaphoreType.DMA,
        ],
    )
    prompts = sc_fn(global_prompt.reshape(POOL_G * CTX_LEN, D),
                    attribute_prompt.reshape(POOL_A * CTX_LEN, D),
                    token_prefix, token_suffix, idxg_rows, idxa_rows)

    tok_nc_fn = pl.pallas_call(
        _tok_nc_kernel,
        grid=(B,),
        in_specs=[
            pl.BlockSpec((CLS, 1, SEQ), lambda b: (0, 0, 0)),
            pl.BlockSpec((POOL_G, CTX_LEN, D), lambda b: (0, 0, 0)),
            pl.BlockSpec((1, 1, D), lambda b: (0, 0, 0)),
            pl.BlockSpec((1, NC_SUF, D), lambda b: (0, 0, 0)),
            pl.BlockSpec((1, SEQ), lambda b: (0, 0)),
        ],
        out_specs=[
            pl.BlockSpec((CLS, 1, SEQ), lambda b: (b, 0, 0)),
            pl.BlockSpec((POOL_G, SEQ, D), lambda b: (0, 0, 0)),
            pl.BlockSpec((POOL_G, SEQ), lambda b: (0, 0)),
        ],
        out_shape=[
            jax.ShapeDtypeStruct((B * CLS, 1, SEQ), jnp.int32),
            jax.ShapeDtypeStruct((POOL_G, SEQ, D), jnp.float32),
            jax.ShapeDtypeStruct((POOL_G, SEQ), jnp.int32),
        ],
    )
    tok3, nc_prompts, nc_tok = tok_nc_fn(
        tokenized_prompts.reshape(CLS, 1, SEQ), global_prompt,
        nc_token_prefix, nc_token_suffix, nc_tokenized_prompts)

    return prompts, tok3.reshape(B * CLS, SEQ), nc_prompts, nc_tok


def kernel(indices_g, indices_a, global_prompt, attribute_prompt,
           token_prefix, token_suffix, tokenized_prompts,
           nc_token_prefix, nc_token_suffix, nc_tokenized_prompts):
    idx_g = indices_g.astype(jnp.int32)
    idx_a = indices_a.astype(jnp.int32)
    return _run(idx_g, idx_a, global_prompt, attribute_prompt,
                token_prefix, token_suffix, tokenized_prompts,
                nc_token_prefix, nc_token_suffix, nc_tokenized_prompts)


# speedup vs baseline: 2.1476x; 2.1476x over previous
import jax
import jax.numpy as jnp
from jax import lax
from jax.experimental import pallas as pl
from jax.experimental.pallas import tpu as pltpu
from jax.experimental.pallas import tpu_sc as plsc

B = 32
CLS = 100
D = 512
CTX_LEN = 12
POOL_G = 10
POOL_A = 100
SEQ = 77
N_CTX = 36
SUF = 40
NC_SUF = 64

NCORE = 2
NSUB = 16
NW = NCORE * NSUB
G_ROWS = B * CTX_LEN
A_ROWS = 2 * B * CTX_LEN
GW = 24
GPT = G_ROWS // GW
APT = A_ROWS // NW

CLS_BLK = 50
NCB = CLS // CLS_BLK


def _sc_gather(g2d_hbm, a2d_hbm, idxg_hbm, idxa_hbm, ctx_hbm,
               idxg_v, idxa_v, gat_g, gat_a, sem):
    cid = lax.axis_index("c")
    sid = lax.axis_index("s")
    w = sid * NCORE + cid

    g_off = pl.multiple_of(GPT * w, 8)
    a_off = pl.multiple_of(APT * w, 8)
    a_dst = pl.multiple_of(G_ROWS + APT * w, 8)

    @pl.when(w < GW)
    def _gather_global():
        pltpu.sync_copy(idxg_hbm.at[pl.ds(g_off, GPT)], idxg_v)
        pltpu.async_copy(g2d_hbm.at[idxg_v], gat_g, sem).wait()
        pltpu.sync_copy(gat_g, ctx_hbm.at[pl.ds(g_off, GPT)])

    pltpu.sync_copy(idxa_hbm.at[pl.ds(a_off, APT)], idxa_v)
    pltpu.async_copy(a2d_hbm.at[idxa_v], gat_a, sem).wait()
    pltpu.sync_copy(gat_a, ctx_hbm.at[pl.ds(a_dst, APT)])


def _prompt_kernel(ctx_ref, pref_ref, suf_ref, tok_ref,
                   out_p_ref, out_t_ref):
    cb = pl.program_id(0)
    b = pl.program_id(1)
    c0 = cb * CLS_BLK

    ctx = ctx_ref[b]
    full = jnp.concatenate([
        pref_ref[pl.ds(c0, CLS_BLK)],
        jnp.broadcast_to(ctx[None], (CLS_BLK, N_CTX, D)),
        suf_ref[pl.ds(c0, CLS_BLK)],
    ], axis=1)
    out_p_ref[...] = full
    out_t_ref[...] = tok_ref[pl.ds(c0, CLS_BLK)]


def _nc_kernel(g_ref, ncp_ref, ncs_ref, nct_ref, out_ncp_ref, out_nct_ref):
    out_ncp_ref[...] = jnp.concatenate([
        jnp.broadcast_to(ncp_ref[...], (POOL_G, 1, D)),
        g_ref[...],
        jnp.broadcast_to(ncs_ref[...], (POOL_G, NC_SUF, D)),
    ], axis=1)
    out_nct_ref[...] = jnp.broadcast_to(nct_ref[...], (POOL_G, SEQ))


@jax.jit
def _run(idx_g, idx_a, global_prompt, attribute_prompt,
         token_prefix, token_suffix, tokenized_prompts,
         nc_token_prefix, nc_token_suffix, nc_tokenized_prompts):
    idxg_rows = (idx_g[:, None] * CTX_LEN
                 + jnp.arange(CTX_LEN, dtype=jnp.int32)[None, :]).reshape(G_ROWS)
    idxa_rows = (idx_a[:, None] * CTX_LEN
                 + jnp.arange(CTX_LEN, dtype=jnp.int32)[None, :]).reshape(A_ROWS)

    sc_fn = pl.kernel(
        _sc_gather,
        out_type=jax.ShapeDtypeStruct((G_ROWS + A_ROWS, D), jnp.float32),
        mesh=plsc.VectorSubcoreMesh(core_axis_name="c", subcore_axis_name="s"),
        scratch_types=[
            pltpu.VMEM((GPT,), jnp.int32),
            pltpu.VMEM((APT,), jnp.int32),
            pltpu.VMEM((GPT, D), jnp.float32),
            pltpu.VMEM((APT, D), jnp.float32),
            pltpu.SemaphoreType.DMA,
        ],
    )
    ctx_all = sc_fn(global_prompt.reshape(POOL_G * CTX_LEN, D),
                    attribute_prompt.reshape(POOL_A * CTX_LEN, D),
                    idxg_rows, idxa_rows)

    main_fn = pl.pallas_call(
        _prompt_kernel,
        grid=(NCB, B),
        in_specs=[
            pl.BlockSpec((B, N_CTX, D), lambda cb, b: (0, 0, 0)),
            pl.BlockSpec((CLS, 1, D), lambda cb, b: (0, 0, 0)),
            pl.BlockSpec((CLS, SUF, D), lambda cb, b: (0, 0, 0)),
            pl.BlockSpec((CLS, 1, SEQ), lambda cb, b: (0, 0, 0)),
        ],
        out_specs=[
            pl.BlockSpec((CLS_BLK, SEQ, D), lambda cb, b: (b * NCB + cb, 0, 0)),
            pl.BlockSpec((CLS_BLK, 1, SEQ), lambda cb, b: (b * NCB + cb, 0, 0)),
        ],
        out_shape=[
            jax.ShapeDtypeStruct((B * CLS, SEQ, D), jnp.float32),
            jax.ShapeDtypeStruct((B * CLS, 1, SEQ), jnp.int32),
        ],
        compiler_params=pltpu.CompilerParams(
            dimension_semantics=("parallel", "parallel")),
    )
    prompts, tok3 = main_fn(ctx_all.reshape(B, N_CTX, D),
                            token_prefix, token_suffix,
                            tokenized_prompts.reshape(CLS, 1, SEQ))

    nc_fn = pl.pallas_call(
        _nc_kernel,
        out_shape=[
            jax.ShapeDtypeStruct((POOL_G, SEQ, D), jnp.float32),
            jax.ShapeDtypeStruct((POOL_G, SEQ), jnp.int32),
        ],
    )
    nc_prompts, nc_tok = nc_fn(global_prompt, nc_token_prefix,
                               nc_token_suffix, nc_tokenized_prompts)

    return prompts, tok3.reshape(B * CLS, SEQ), nc_prompts, nc_tok


def kernel(indices_g, indices_a, global_prompt, attribute_prompt,
           token_prefix, token_suffix, tokenized_prompts,
           nc_token_prefix, nc_token_suffix, nc_tokenized_prompts):
    idx_g = indices_g.astype(jnp.int32)
    idx_a = indices_a.astype(jnp.int32)
    return _run(idx_g, idx_a, global_prompt, attribute_prompt,
                token_prefix, token_suffix, tokenized_prompts,
                nc_token_prefix, nc_token_suffix, nc_tokenized_prompts)
